# Initial kernel scaffold; baseline (speedup 1.0000x reference)
import jax
import jax.numpy as jnp
from jax.experimental import pallas as pl


def kernel(x, W, b, tables):
    raise NotImplementedError("write your pallas kernel here")


# speedup vs baseline: 1.0314x; 1.0314x over previous
import functools

import jax
import jax.numpy as jnp
from jax import lax
from jax.experimental import pallas as pl
from jax.experimental.pallas import tpu as pltpu
from jax.experimental.pallas import tpu_sc as plsc

_N_NUM_IN = 13
_N_NUM_OUT = 32
_N_FIELDS = 26
_VOCAB = 100000
_EMB_DIM = 32
_BATCH = 16384
_N_COLS = _N_NUM_IN + _N_FIELDS
_OUT_DIM = _N_NUM_OUT + _N_FIELDS * _EMB_DIM

_NW = 32
_B_PER_W = _BATCH // _NW
_L = 16


@functools.partial(
    pl.kernel,
    out_type=jax.ShapeDtypeStruct((_BATCH, _OUT_DIM), jnp.float32),
    mesh=plsc.VectorSubcoreMesh(core_axis_name="c", subcore_axis_name="s"),
    scratch_types=[
        pltpu.VMEM((_B_PER_W, _N_COLS), jnp.float32),
        pltpu.VMEM((_N_NUM_IN, _N_NUM_OUT), jnp.float32),
        pltpu.VMEM((_N_NUM_OUT,), jnp.float32),
        pltpu.VMEM((_N_FIELDS, _B_PER_W), jnp.int32),
        pltpu.VMEM((_B_PER_W, _N_NUM_OUT), jnp.float32),
        pltpu.VMEM((_B_PER_W, _EMB_DIM), jnp.float32),
        pltpu.SemaphoreType.DMA,
    ],
    compiler_params=pltpu.CompilerParams(
        use_tc_tiling_on_sc=False, needs_layout_passes=False),
)
def _cond_embed(x_hbm, wt_hbm, b_hbm, tab_hbm, out_hbm,
                xv, wv, bv, idx_v, onum, gbuf, gsem):
    wid = lax.axis_index("s") * 2 + lax.axis_index("c")
    base = wid * _B_PER_W

    pltpu.sync_copy(x_hbm.at[pl.ds(base, _B_PER_W), :], xv)
    pltpu.sync_copy(wt_hbm, wv)
    pltpu.sync_copy(b_hbm, bv)

    lanes = lax.iota(jnp.int32, 16)

    for f in range(_N_FIELDS):
        col = jnp.full((16,), _N_NUM_IN + f, jnp.int32)
        off = f * _VOCAB

        def jbody(j, carry, col=col, off=off, f=f):
            r0 = j * _L
            rows = r0 + lanes
            vals = plsc.load_gather(xv, [rows, col])
            idx_v[f, pl.ds(r0, _L)] = vals.astype(jnp.int32) + off
            return carry

        lax.fori_loop(0, _B_PER_W // _L, jbody, 0)

    w_lo = [wv[k, pl.ds(0, _L)] for k in range(_N_NUM_IN)]
    w_hi = [wv[k, pl.ds(_L, _L)] for k in range(_N_NUM_IN)]
    b_lo = bv[pl.ds(0, _L)]
    b_hi = bv[pl.ds(_L, _L)]

    def rbody(r, carry):
        acc0 = b_lo
        acc1 = b_hi
        vrow = xv[r, pl.ds(0, _L)]
        for k in range(_N_NUM_IN):
            s = vrow[k]
            acc0 = acc0 + s * w_lo[k]
            acc1 = acc1 + s * w_hi[k]
        onum[r, pl.ds(0, _L)] = acc0
        onum[r, pl.ds(_L, _L)] = acc1
        return carry

    lax.fori_loop(0, _B_PER_W, rbody, 0)
    pltpu.sync_copy(onum, out_hbm.at[pl.ds(base, _B_PER_W), pl.ds(0, _N_NUM_OUT)])

    def fbody(f, carry):
        pltpu.async_copy(tab_hbm.at[idx_v.at[f]], gbuf, gsem).wait()
        pltpu.sync_copy(
            gbuf,
            out_hbm.at[pl.ds(base, _B_PER_W),
                       pl.ds(_N_NUM_OUT + _EMB_DIM * f, _EMB_DIM)])
        return carry

    lax.fori_loop(0, _N_FIELDS, fbody, 0)


def kernel(x, W, b, tables):
    tab = tables.reshape(_N_FIELDS * _VOCAB, _EMB_DIM)
    return _cond_embed(x, W.T, b, tab)


# speedup vs baseline: 1.5415x; 1.4946x over previous
import functools

import jax
import jax.numpy as jnp
from jax import lax
from jax.experimental import pallas as pl
from jax.experimental.pallas import tpu as pltpu
from jax.experimental.pallas import tpu_sc as plsc

_N_NUM_IN = 13
_N_NUM_OUT = 32
_N_FIELDS = 26
_VOCAB = 100000
_EMB_DIM = 32
_BATCH = 16384
_OUT_DIM = _N_NUM_OUT + _N_FIELDS * _EMB_DIM

_NW = 32
_ROWS_PER_W = 26
_HALF = _BATCH // 2
_L = 16


@functools.partial(
    pl.kernel,
    out_type=jax.ShapeDtypeStruct((_OUT_DIM, _BATCH), jnp.float32),
    mesh=plsc.VectorSubcoreMesh(core_axis_name="c", subcore_axis_name="s"),
    scratch_types=[
        pltpu.VMEM((_VOCAB,), jnp.float32),
        pltpu.VMEM((_BATCH,), jnp.int32),
        pltpu.VMEM((_HALF,), jnp.float32),
        pltpu.VMEM((_N_NUM_OUT, _L), jnp.float32),
        pltpu.VMEM((_N_NUM_OUT,), jnp.float32),
    ],
    compiler_params=pltpu.CompilerParams(
        use_tc_tiling_on_sc=False, needs_layout_passes=False),
)
def _cond_embed(xt_hbm, w_hbm, b_hbm, tab_hbm, out_hbm,
                rowv, idxv, outb, wv, bv):
    wid = lax.axis_index("s") * 2 + lax.axis_index("c")

    pltpu.sync_copy(w_hbm, wv)
    pltpu.sync_copy(b_hbm, bv)
    osplat = jnp.zeros((_L,), jnp.int32) + wid
    wvecs = [
        plsc.load_gather(wv, [osplat, jnp.full((_L,), k, jnp.int32)])
        for k in range(_N_NUM_IN)
    ]
    bvec = plsc.load_gather(bv, [osplat])
    lch = 4096
    for c in range(_BATCH // lch):
        for k in range(_N_NUM_IN):
            pltpu.sync_copy(xt_hbm.at[k, pl.ds(c * lch, lch)],
                            rowv.at[pl.ds(k * lch, lch)])

        def lin_body(j, carry):
            acc = bvec
            for k in range(_N_NUM_IN):
                acc = acc + wvecs[k] * rowv[pl.ds(k * lch + j * _L, _L)]
            outb[pl.ds(j * _L, _L)] = acc
            return carry

        lax.fori_loop(0, lch // _L, lin_body, 0)
        pltpu.sync_copy(outb.at[pl.ds(0, lch)],
                        out_hbm.at[wid, pl.ds(c * lch, lch)])

    def load_field_idx(f):
        pltpu.sync_copy(xt_hbm.at[_N_NUM_IN + f], rowv.at[pl.ds(0, _BATCH)])

        def conv(j, carry):
            idxv[pl.ds(j * _L, _L)] = rowv[pl.ds(j * _L, _L)].astype(jnp.int32)
            return carry

        lax.fori_loop(0, _BATCH // _L, conv, 0)

    for i in range(_ROWS_PER_W):
        r = wid * _ROWS_PER_W + i
        f = r // _EMB_DIM
        if i == 0:
            load_field_idx(f)
        else:
            @pl.when(r % _EMB_DIM == 0)
            def _(f=f):
                load_field_idx(f)

        pltpu.sync_copy(tab_hbm.at[r], rowv)
        for c in range(2):
            def gbody(j, carry, c=c):
                iv = idxv[pl.ds(c * _HALF + j * _L, _L)]
                outb[pl.ds(j * _L, _L)] = plsc.load_gather(rowv, [iv])
                return carry

            lax.fori_loop(0, _HALF // _L, gbody, 0)
            pltpu.sync_copy(outb,
                            out_hbm.at[_N_NUM_OUT + r, pl.ds(c * _HALF, _HALF)])


def kernel(x, W, b, tables):
    xt = x.T
    wp = jnp.pad(W, ((0, 0), (0, _L - _N_NUM_IN)))
    tab = tables.transpose(0, 2, 1).reshape(_N_FIELDS * _EMB_DIM, _VOCAB)
    out_t = _cond_embed(xt, wp, b, tab)
    return out_t.T


# speedup vs baseline: 1.5689x; 1.0177x over previous
import functools

import jax
import jax.numpy as jnp
from jax import lax
from jax.experimental import pallas as pl
from jax.experimental.pallas import tpu as pltpu
from jax.experimental.pallas import tpu_sc as plsc

_N_NUM_IN = 13
_N_NUM_OUT = 32
_N_FIELDS = 26
_VOCAB = 100000
_EMB_DIM = 32
_BATCH = 16384
_OUT_DIM = _N_NUM_OUT + _N_FIELDS * _EMB_DIM

_NW = 32
_ROWS_PER_W = 26
_QB = _BATCH // 4
_L = 16


@functools.partial(
    pl.kernel,
    out_type=jax.ShapeDtypeStruct((_OUT_DIM, _BATCH), jnp.float32),
    mesh=plsc.VectorSubcoreMesh(core_axis_name="c", subcore_axis_name="s"),
    scratch_types=[
        pltpu.VMEM((_VOCAB,), jnp.float32),
        pltpu.VMEM((_BATCH,), jnp.int32),
        pltpu.VMEM((_QB,), jnp.float32),
        pltpu.VMEM((_QB,), jnp.float32),
        pltpu.VMEM((_N_NUM_OUT, _L), jnp.float32),
        pltpu.VMEM((_N_NUM_OUT,), jnp.float32),
        pltpu.SemaphoreType.DMA,
        pltpu.SemaphoreType.DMA,
        pltpu.SemaphoreType.DMA,
    ],
    compiler_params=pltpu.CompilerParams(
        use_tc_tiling_on_sc=False, needs_layout_passes=False),
)
def _cond_embed(xt_hbm, w_hbm, b_hbm, tab_hbm, out_hbm,
                rowv, idxv, outb0, outb1, wv, bv, rsem, wsem0, wsem1):
    wid = lax.axis_index("s") * 2 + lax.axis_index("c")
    outbs = (outb0, outb1)
    wsems = (wsem0, wsem1)
    wh = [None, None]

    def flush(q, src_row_hbm_slice):
        wh[q % 2] = pltpu.async_copy(outbs[q % 2], src_row_hbm_slice,
                                     wsems[q % 2])

    def drain(q):
        if wh[q % 2] is not None:
            wh[q % 2].wait()
            wh[q % 2] = None

    pltpu.sync_copy(w_hbm, wv)
    pltpu.sync_copy(b_hbm, bv)
    osplat = jnp.zeros((_L,), jnp.int32) + wid
    wvecs = [
        plsc.load_gather(wv, [osplat, jnp.full((_L,), k, jnp.int32)])
        for k in range(_N_NUM_IN)
    ]
    bvec = plsc.load_gather(bv, [osplat])
    for c in range(4):
        for k in range(_N_NUM_IN):
            pltpu.sync_copy(xt_hbm.at[k, pl.ds(c * _QB, _QB)],
                            rowv.at[pl.ds(k * _QB, _QB)])
        drain(c)
        ob = outbs[c % 2]

        def lin_body(j, carry, ob=ob):
            acc = bvec
            for k in range(_N_NUM_IN):
                acc = acc + wvecs[k] * rowv[pl.ds(k * _QB + j * _L, _L)]
            ob[pl.ds(j * _L, _L)] = acc
            return carry

        lax.fori_loop(0, _QB // _L, lin_body, 0)
        flush(c, out_hbm.at[wid, pl.ds(c * _QB, _QB)])
    drain(0)
    drain(1)

    def load_field_idx(f):
        pltpu.sync_copy(xt_hbm.at[_N_NUM_IN + f], rowv.at[pl.ds(0, _BATCH)])

        def conv(j, carry):
            idxv[pl.ds(j * _L, _L)] = (
                rowv[pl.ds(j * _L, _L)].astype(jnp.int32))
            return carry

        lax.fori_loop(0, _BATCH // _L, conv, 0)

    row0 = wid * _ROWS_PER_W
    load_field_idx(row0 // _EMB_DIM)
    rh = pltpu.async_copy(tab_hbm.at[row0], rowv, rsem)

    for i in range(_ROWS_PER_W):
        r = row0 + i
        rh.wait()
        for q in range(4):
            drain(q)
            ob = outbs[q % 2]

            def gq(j, carry, ob=ob, q=q):
                iv = idxv[pl.ds(q * _QB + j * _L, _L)]
                ob[pl.ds(j * _L, _L)] = plsc.load_gather(rowv, [iv])
                return carry

            lax.fori_loop(0, _QB // _L, gq, 0)
            if q < 3:
                flush(q, out_hbm.at[_N_NUM_OUT + r, pl.ds(q * _QB, _QB)])
        if i + 1 < _ROWS_PER_W:
            @pl.when((r + 1) % _EMB_DIM == 0)
            def _(r=r):
                load_field_idx((r + 1) // _EMB_DIM)
            rh = pltpu.async_copy(tab_hbm.at[r + 1], rowv, rsem)
        flush(3, out_hbm.at[_N_NUM_OUT + r, pl.ds(3 * _QB, _QB)])

    drain(0)
    drain(1)


def kernel(x, W, b, tables):
    xt = x.T
    wp = jnp.pad(W, ((0, 0), (0, _L - _N_NUM_IN)))
    tab = tables.transpose(0, 2, 1).reshape(_N_FIELDS * _EMB_DIM, _VOCAB)
    out_t = _cond_embed(xt, wp, b, tab)
    return out_t.T
